# Initial kernel scaffold; baseline (speedup 1.0000x reference)
import jax
import jax.numpy as jnp
from jax.experimental import pallas as pl


def kernel(x_nchw, w1, b1, w2, b2):
    raise NotImplementedError("write your pallas kernel here")


# speedup vs baseline: 1.0573x; 1.0573x over previous
import functools

import jax
import jax.numpy as jnp
from jax import lax
from jax.experimental import pallas as pl
from jax.experimental.pallas import tpu as pltpu

KS = 3
PAD = 128


def _resblock_kernel(x_ref, w1_ref, b1_ref, w2_ref, b2_ref, o_ref, pad_ref,
                     *, H, W):
    R = x_ref.shape[1]
    HW = H * W
    bf16 = jnp.bfloat16

    col = lax.broadcasted_iota(jnp.int32, (1, HW), 1) % W
    ok_left = col != 0
    ok_right = col != (W - 1)

    zeros_border = jnp.zeros((R, PAD), bf16)
    pad_ref[:, :PAD] = zeros_border
    pad_ref[:, PAD + HW:] = zeros_border

    x32 = x_ref[0]

    def conv3x3(w_ref, b_ref):
        acc = jnp.zeros((R, HW), jnp.float32)
        for dy in range(KS):
            for dx in range(KS):
                off = (dy - 1) * W + (dx - 1)
                patch = pad_ref[:, PAD + off:PAD + off + HW]
                if dx == 0:
                    patch = jnp.where(ok_left, patch, bf16(0))
                elif dx == 2:
                    patch = jnp.where(ok_right, patch, bf16(0))
                acc = acc + jnp.dot(w_ref[dy * KS + dx], patch,
                                    preferred_element_type=jnp.float32)
        return acc + b_ref[...]

    pad_ref[:, PAD:PAD + HW] = x32.astype(bf16)
    h = jnp.maximum(conv3x3(w1_ref, b1_ref), 0.0)

    pad_ref[:, PAD:PAD + HW] = h.astype(bf16)
    y = conv3x3(w2_ref, b2_ref) + x32
    o_ref[0] = jnp.maximum(y, 0.0).astype(o_ref.dtype)


def _batch_group(N, C, max_rows=128):
    best = 1
    for g in range(1, min(N, max(1, max_rows // C)) + 1):
        if N % g == 0 and (g == N or (g * C) % 8 == 0):
            best = g
    return best


def kernel(x_nchw, w1, b1, w2, b2):
    N, Cin, H, W = x_nchw.shape
    Cout = w2.shape[-1]
    assert Cout == Cin
    assert PAD >= W + 1
    HW = H * W

    G = _batch_group(N, Cin)
    NB = N // G
    R = G * Cin

    x_blk = x_nchw.reshape(NB, R, HW)

    eyeg = jnp.eye(G, dtype=jnp.float32)
    w1_tap = jnp.transpose(w1, (0, 1, 3, 2)).reshape(KS * KS, Cin, Cin)
    w2_tap = jnp.transpose(w2, (0, 1, 3, 2)).reshape(KS * KS, Cout, Cin)
    w1_bd = jnp.einsum('gh,toc->tgohc', eyeg, w1_tap).reshape(KS * KS, R, R)
    w2_bd = jnp.einsum('gh,toc->tgohc', eyeg, w2_tap).reshape(KS * KS, R, R)
    w1_bd = w1_bd.astype(jnp.bfloat16)
    w2_bd = w2_bd.astype(jnp.bfloat16)
    b1_bd = jnp.tile(b1.reshape(Cin), (G,)).reshape(R, 1).astype(jnp.float32)
    b2_bd = jnp.tile(b2.reshape(Cout), (G,)).reshape(R, 1).astype(jnp.float32)

    body = functools.partial(_resblock_kernel, H=H, W=W)

    out_blk = pl.pallas_call(
        body,
        out_shape=jax.ShapeDtypeStruct((NB, R, HW), x_nchw.dtype),
        grid_spec=pltpu.PrefetchScalarGridSpec(
            num_scalar_prefetch=0,
            grid=(NB,),
            in_specs=[
                pl.BlockSpec((1, R, HW), lambda n: (n, 0, 0)),
                pl.BlockSpec((KS * KS, R, R), lambda n: (0, 0, 0)),
                pl.BlockSpec((R, 1), lambda n: (0, 0)),
                pl.BlockSpec((KS * KS, R, R), lambda n: (0, 0, 0)),
                pl.BlockSpec((R, 1), lambda n: (0, 0)),
            ],
            out_specs=pl.BlockSpec((1, R, HW), lambda n: (n, 0, 0)),
            scratch_shapes=[pltpu.VMEM((R, HW + 2 * PAD), jnp.bfloat16)],
        ),
        compiler_params=pltpu.CompilerParams(
            dimension_semantics=("parallel",)),
    )(x_blk, w1_bd, b1_bd, w2_bd, b2_bd)

    return out_blk.reshape(N, Cout, H, W)


# speedup vs baseline: 1.5569x; 1.4725x over previous
import functools

import jax
import jax.numpy as jnp
from jax import lax
from jax.experimental import pallas as pl
from jax.experimental.pallas import tpu as pltpu

KS = 3
PAD = 128


def _resblock_kernel(x_ref, w1_ref, b1_ref, w2_ref, b2_ref, o_ref, pad_ref,
                     *, H, W):
    G, C = x_ref.shape[0], x_ref.shape[1]
    R = G * C
    HW = H * W
    bf16 = jnp.bfloat16

    col = lax.broadcasted_iota(jnp.int32, (1, HW), 1) % W
    ok_left = col != 0
    ok_right = col != (W - 1)

    zeros_border = jnp.zeros((R, PAD), bf16)
    pad_ref[:, :PAD] = zeros_border
    pad_ref[:, PAD + HW:] = zeros_border

    x32 = x_ref[...].reshape(R, HW)

    def conv3x3(w_ref, b_ref):
        acc = jnp.zeros((R, HW), jnp.float32)
        for dy in range(KS):
            for dx in range(KS):
                off = (dy - 1) * W + (dx - 1)
                patch = pad_ref[:, PAD + off:PAD + off + HW]
                if dx == 0:
                    patch = jnp.where(ok_left, patch, bf16(0))
                elif dx == 2:
                    patch = jnp.where(ok_right, patch, bf16(0))
                acc = acc + jnp.dot(w_ref[dy * KS + dx], patch,
                                    preferred_element_type=jnp.float32)
        return acc + b_ref[...]

    pad_ref[:, PAD:PAD + HW] = x32.astype(bf16)
    h = jnp.maximum(conv3x3(w1_ref, b1_ref), 0.0)

    pad_ref[:, PAD:PAD + HW] = h.astype(bf16)
    y = jnp.maximum(conv3x3(w2_ref, b2_ref) + x32, 0.0)
    o_ref[...] = y.reshape(G, C, H, W)


def _batch_group(N, C, max_rows=128):
    best = 1
    for g in range(1, min(N, max(1, max_rows // C)) + 1):
        if N % g == 0 and (g == N or (g * C) % 8 == 0):
            best = g
    return best


def kernel(x_nchw, w1, b1, w2, b2):
    N, Cin, H, W = x_nchw.shape
    Cout = w2.shape[-1]
    assert Cout == Cin
    assert PAD >= W + 1

    G = _batch_group(N, Cin)
    NB = N // G
    R = G * Cin

    eyeg = jnp.eye(G, dtype=jnp.float32)
    w1_tap = jnp.transpose(w1, (0, 1, 3, 2)).reshape(KS * KS, Cin, Cin)
    w2_tap = jnp.transpose(w2, (0, 1, 3, 2)).reshape(KS * KS, Cout, Cin)
    w1_bd = jnp.einsum('gh,toc->tgohc', eyeg, w1_tap).reshape(KS * KS, R, R)
    w2_bd = jnp.einsum('gh,toc->tgohc', eyeg, w2_tap).reshape(KS * KS, R, R)
    w1_bd = w1_bd.astype(jnp.bfloat16)
    w2_bd = w2_bd.astype(jnp.bfloat16)
    b1_bd = jnp.tile(b1.reshape(Cin), (G,)).reshape(R, 1).astype(jnp.float32)
    b2_bd = jnp.tile(b2.reshape(Cout), (G,)).reshape(R, 1).astype(jnp.float32)

    body = functools.partial(_resblock_kernel, H=H, W=W)

    out = pl.pallas_call(
        body,
        out_shape=jax.ShapeDtypeStruct((N, Cin, H, W), x_nchw.dtype),
        grid_spec=pltpu.PrefetchScalarGridSpec(
            num_scalar_prefetch=0,
            grid=(NB,),
            in_specs=[
                pl.BlockSpec((G, Cin, H, W), lambda n: (n, 0, 0, 0)),
                pl.BlockSpec((KS * KS, R, R), lambda n: (0, 0, 0)),
                pl.BlockSpec((R, 1), lambda n: (0, 0)),
                pl.BlockSpec((KS * KS, R, R), lambda n: (0, 0, 0)),
                pl.BlockSpec((R, 1), lambda n: (0, 0)),
            ],
            out_specs=pl.BlockSpec((G, Cin, H, W), lambda n: (n, 0, 0, 0)),
            scratch_shapes=[pltpu.VMEM((R, H * W + 2 * PAD), jnp.bfloat16)],
        ),
        compiler_params=pltpu.CompilerParams(
            dimension_semantics=("parallel",)),
    )(x_nchw, w1_bd, b1_bd, w2_bd, b2_bd)

    return out


# speedup vs baseline: 1.8605x; 1.1950x over previous
import functools

import jax
import jax.numpy as jnp
from jax import lax
from jax.experimental import pallas as pl
from jax.experimental.pallas import tpu as pltpu

KS = 3
PAD = 128


def _resblock_kernel(x_ref, w1_ref, b1_ref, w2_ref, b2_ref,
                     o_ref, sc_ref, sl_ref, sr_ref, *, H, W):
    C = x_ref.shape[2]
    L = H * 2 * W
    bf16 = jnp.bfloat16

    lmod = lax.broadcasted_iota(jnp.int32, (1, L), 1) % W
    ok_left = lmod != 0
    ok_right = lmod != (W - 1)

    zb = jnp.zeros((C, PAD), bf16)
    for s in (sc_ref, sl_ref, sr_ref):
        s[:, :PAD] = zb
        s[:, PAD + L:] = zb

    def load_shifted():
        left = jnp.where(ok_left, sc_ref[:, PAD - 1:PAD - 1 + L], bf16(0))
        right = jnp.where(ok_right, sc_ref[:, PAD + 1:PAD + 1 + L], bf16(0))
        sl_ref[:, PAD:PAD + L] = left
        sr_ref[:, PAD:PAD + L] = right

    def conv3x3(w_ref, b_ref):
        acc = jnp.zeros((C, L), jnp.float32)
        for dy in range(KS):
            base = PAD + (dy - 1) * 2 * W
            for dx, s_ref in ((0, sl_ref), (1, sc_ref), (2, sr_ref)):
                acc = acc + jnp.dot(w_ref[dy * KS + dx],
                                    s_ref[:, base:base + L],
                                    preferred_element_type=jnp.float32)
        return acc + b_ref[...]

    x_pk = jnp.concatenate([x_ref[0, 0], x_ref[0, 1]], axis=2).reshape(C, L)

    sc_ref[:, PAD:PAD + L] = x_pk.astype(bf16)
    load_shifted()
    h = jnp.maximum(conv3x3(w1_ref, b1_ref), 0.0)

    sc_ref[:, PAD:PAD + L] = h.astype(bf16)
    load_shifted()
    y = jnp.maximum(conv3x3(w2_ref, b2_ref) + x_pk, 0.0)

    y3 = y.reshape(C, H, 2 * W)
    o_ref[0, 0] = y3[:, :, :W]
    o_ref[0, 1] = y3[:, :, W:]


def kernel(x_nchw, w1, b1, w2, b2):
    N, C, H, W = x_nchw.shape
    assert w2.shape[-1] == C and N % 2 == 0 and 2 * W % 128 == 0
    assert PAD >= 2 * W

    NB = N // 2
    x_pairs = x_nchw.reshape(NB, 2, C, H, W)

    w1_tap = jnp.transpose(w1, (0, 1, 3, 2)).reshape(KS * KS, C, C)
    w2_tap = jnp.transpose(w2, (0, 1, 3, 2)).reshape(KS * KS, C, C)
    w1_tap = w1_tap.astype(jnp.bfloat16)
    w2_tap = w2_tap.astype(jnp.bfloat16)
    b1_c = b1.reshape(C, 1).astype(jnp.float32)
    b2_c = b2.reshape(C, 1).astype(jnp.float32)

    body = functools.partial(_resblock_kernel, H=H, W=W)
    pair_spec = pl.BlockSpec((1, 2, C, H, W), lambda n: (n, 0, 0, 0, 0))
    span = H * 2 * W + 2 * PAD

    out = pl.pallas_call(
        body,
        out_shape=jax.ShapeDtypeStruct((NB, 2, C, H, W), x_nchw.dtype),
        grid_spec=pltpu.PrefetchScalarGridSpec(
            num_scalar_prefetch=0,
            grid=(NB,),
            in_specs=[
                pair_spec,
                pl.BlockSpec((KS * KS, C, C), lambda n: (0, 0, 0)),
                pl.BlockSpec((C, 1), lambda n: (0, 0)),
                pl.BlockSpec((KS * KS, C, C), lambda n: (0, 0, 0)),
                pl.BlockSpec((C, 1), lambda n: (0, 0)),
            ],
            out_specs=pair_spec,
            scratch_shapes=[pltpu.VMEM((C, span), jnp.bfloat16)] * 3,
        ),
        compiler_params=pltpu.CompilerParams(
            dimension_semantics=("parallel",)),
    )(x_pairs, w1_tap, b1_c, w2_tap, b2_c)

    return out.reshape(N, C, H, W)


# speedup vs baseline: 3.0131x; 1.6195x over previous
import functools

import jax
import jax.numpy as jnp
from jax import lax
from jax.experimental import pallas as pl
from jax.experimental.pallas import tpu as pltpu

KS = 3
PAD = 128
NCHUNK = 4


def _resblock_kernel(x_ref, w1_ref, b1_ref, w2_ref, b2_ref,
                     o_ref, s1_ref, s2_ref, sx_ref, *, H, W):
    C = x_ref.shape[2]
    W2 = 2 * W
    L = H * W2
    CH = L // NCHUNK
    HH = CH // W2
    bf16 = jnp.bfloat16

    lmod = lax.broadcasted_iota(jnp.int32, (1, CH), 1) % W
    ok_left = lmod != 0
    ok_right = lmod != (W - 1)

    zb = jnp.zeros((3 * C, PAD), bf16)
    for s in (s1_ref, s2_ref):
        s[:, :PAD] = zb
        s[:, PAD + L:] = zb

    def build_shifted(s_ref):
        for j in range(NCHUNK):
            lo = PAD + j * CH
            left = jnp.where(ok_left, s_ref[C:2 * C, lo - 1:lo - 1 + CH],
                             bf16(0))
            right = jnp.where(ok_right, s_ref[C:2 * C, lo + 1:lo + 1 + CH],
                              bf16(0))
            s_ref[:C, lo:lo + CH] = left
            s_ref[2 * C:, lo:lo + CH] = right

    def conv_chunk(w_ref, b_ref, s_ref, j):
        lo = PAD + j * CH
        acc = jnp.zeros((C, CH), jnp.float32)
        for dy in range(KS):
            off = (dy - 1) * W2
            acc = acc + jnp.dot(w_ref[dy], s_ref[:, lo + off:lo + off + CH],
                                preferred_element_type=jnp.float32)
        return acc + b_ref[...]

    for j in range(NCHUNK):
        xa = x_ref[0, 0, :, j * HH:(j + 1) * HH, :]
        xb = x_ref[0, 1, :, j * HH:(j + 1) * HH, :]
        xpk = jnp.concatenate([xa, xb], axis=2).reshape(C, CH)
        sx_ref[:, j * CH:(j + 1) * CH] = xpk
        s1_ref[C:2 * C, PAD + j * CH:PAD + (j + 1) * CH] = xpk.astype(bf16)

    build_shifted(s1_ref)
    for j in range(NCHUNK):
        h = jnp.maximum(conv_chunk(w1_ref, b1_ref, s1_ref, j), 0.0)
        s2_ref[C:2 * C, PAD + j * CH:PAD + (j + 1) * CH] = h.astype(bf16)

    build_shifted(s2_ref)
    for j in range(NCHUNK):
        y = conv_chunk(w2_ref, b2_ref, s2_ref, j)
        y = jnp.maximum(y + sx_ref[:, j * CH:(j + 1) * CH], 0.0)
        y3 = y.reshape(C, HH, W2)
        o_ref[0, 0, :, j * HH:(j + 1) * HH, :] = y3[:, :, :W]
        o_ref[0, 1, :, j * HH:(j + 1) * HH, :] = y3[:, :, W:]


def kernel(x_nchw, w1, b1, w2, b2):
    N, C, H, W = x_nchw.shape
    assert w2.shape[-1] == C and N % 2 == 0 and 2 * W % 128 == 0
    assert PAD >= 2 * W and (H * 2 * W) % (NCHUNK * 2 * W) == 0

    NB = N // 2
    x_pairs = x_nchw.reshape(NB, 2, C, H, W)

    def stack_w(w):
        t = jnp.transpose(w, (0, 1, 3, 2))
        t = jnp.transpose(t, (0, 2, 1, 3))
        return t.reshape(KS, C, KS * C).astype(jnp.bfloat16)

    w1_s = stack_w(w1)
    w2_s = stack_w(w2)
    b1_c = b1.reshape(C, 1).astype(jnp.float32)
    b2_c = b2.reshape(C, 1).astype(jnp.float32)

    body = functools.partial(_resblock_kernel, H=H, W=W)
    pair_spec = pl.BlockSpec((1, 2, C, H, W), lambda n: (n, 0, 0, 0, 0))
    span = H * 2 * W + 2 * PAD

    out = pl.pallas_call(
        body,
        out_shape=jax.ShapeDtypeStruct((NB, 2, C, H, W), x_nchw.dtype),
        grid_spec=pltpu.PrefetchScalarGridSpec(
            num_scalar_prefetch=0,
            grid=(NB,),
            in_specs=[
                pair_spec,
                pl.BlockSpec((KS, C, KS * C), lambda n: (0, 0, 0)),
                pl.BlockSpec((C, 1), lambda n: (0, 0)),
                pl.BlockSpec((KS, C, KS * C), lambda n: (0, 0, 0)),
                pl.BlockSpec((C, 1), lambda n: (0, 0)),
            ],
            out_specs=pair_spec,
            scratch_shapes=[
                pltpu.VMEM((3 * C, span), jnp.bfloat16),
                pltpu.VMEM((3 * C, span), jnp.bfloat16),
                pltpu.VMEM((C, H * 2 * W), jnp.float32),
            ],
        ),
        compiler_params=pltpu.CompilerParams(
            dimension_semantics=("parallel",)),
    )(x_pairs, w1_s, b1_c, w2_s, b2_c)

    return out.reshape(N, C, H, W)
